# baseline (speedup 1.0000x reference)
import functools

import jax
import jax.numpy as jnp
from jax import lax
from jax.experimental import pallas as pl
from jax.experimental.pallas import tpu as pltpu
from jax.experimental.pallas import tpu_sc as plsc

RATIO = 4


def _entropy_topk_body(attn_ref, gidx_ref, ent_s, idx_s):
    h = pl.program_id(0)
    k = pl.program_id(1)
    nh = pl.num_programs(0) - 1
    nk = pl.num_programs(1)

    @pl.when(h < nh)
    def _entropy():
        a = attn_ref[...]
        e = (jnp.exp(a) * a)[0]
        lb, s = e.shape
        acc = e[:, 0:128]
        for t in range(1, s // 128):
            acc = acc + e[:, t * 128:(t + 1) * 128]
        tp = acc.T.reshape(16, 8, lb)
        p = tp[0]
        for t in range(1, 16):
            p = p + tp[t]
        t1 = p[0:4] + p[4:8]
        t2 = t1[0:2] + t1[2:4]
        v = t2[0:1] + t2[1:2]
        ent_s[pl.ds(h % 2, 1), pl.ds(k, 1), :] = v.reshape(1, 1, v.shape[-1])

    @pl.when(h >= 1)
    def _topk_chunk():
        hh = h - 1
        rm = (h + 1) % 2
        nck, lb = ent_s.shape[1], ent_s.shape[2]
        kk = gidx_ref.shape[-1]
        ci = k
        v8 = ent_s[pl.ds(rm, 1)][0]
        vi_col = ent_s[pl.ds(rm, 1), pl.ds(ci, 1), :].reshape(lb, 1)
        cntmat = None
        jj = lax.broadcasted_iota(jnp.int32, (lb, lb), 1)
        ii = lax.broadcasted_iota(jnp.int32, (lb, lb), 0)
        intra = jj < ii
        for cj in range(nck):
            vj = v8[cj:cj + 1, :]
            gt = vj > vi_col
            eq = vj == vi_col
            tiem = (eq & (cj < ci)) | (eq & (cj == ci) & intra)
            m = gt | tiem
            mf = jnp.where(m, 1.0, 0.0)
            cntmat = mf if cntmat is None else cntmat + mf
        cnt = jnp.sum(cntmat, axis=1, keepdims=True)
        rr = lax.broadcasted_iota(jnp.int32, (lb, kk), 1).astype(jnp.float32)
        onehot = cnt == rr
        gi = (lax.broadcasted_iota(jnp.int32, (lb, kk), 0).astype(jnp.float32)
              + ci.astype(jnp.float32) * lb)
        contrib = jnp.sum(jnp.where(onehot, gi, 0.0), axis=0, keepdims=True)

        @pl.when(k == 0)
        def _():
            idx_s[...] = contrib

        @pl.when(k > 0)
        def _():
            idx_s[...] = idx_s[...] + contrib

        @pl.when(k == nk - 1)
        def _():
            gidx = idx_s[...].astype(jnp.int32) * 16 + hh
            gidx_ref[...] = gidx.reshape(gidx_ref.shape)


def _entropy_topk(attn3, num_output):
    nh, l, s = attn3.shape
    lb = 256
    nk = l // lb

    def attn_map(h, k):
        return (jnp.minimum(h, nh - 1), jnp.where(h == nh, nk - 1, k), 0)

    def out_map(h, k):
        return (jnp.maximum(h - 1, 0), 0, 0)

    return pl.pallas_call(
        _entropy_topk_body,
        grid=(nh + 1, nk),
        in_specs=[pl.BlockSpec((1, lb, s), attn_map)],
        out_specs=pl.BlockSpec((1, 1, num_output), out_map),
        out_shape=jax.ShapeDtypeStruct((nh, 1, num_output), jnp.int32),
        scratch_shapes=[
            pltpu.VMEM((2, nk, lb), jnp.float32),
            pltpu.VMEM((1, num_output), jnp.float32),
        ],
    )(attn3)


def _sc_gather(xf, cf, gidx, num_rows):
    mesh = plsc.VectorSubcoreMesh(core_axis_name="c", subcore_axis_name="s")
    info = plsc.get_sparse_core_info()
    nw = info.num_cores * info.num_subcores
    rows_per_w = num_rows // nw
    cols_per_w = rows_per_w // 16

    @functools.partial(
        pl.kernel,
        out_type=(
            jax.ShapeDtypeStruct((num_rows, 64), jnp.float32),
            jax.ShapeDtypeStruct((num_rows, 64), jnp.float32),
        ),
        mesh=mesh,
        compiler_params=pltpu.CompilerParams(use_tc_tiling_on_sc=False),
        scratch_types=[
            pltpu.VMEM((rows_per_w,), jnp.int32),
            pltpu.VMEM((rows_per_w, 64), jnp.float32),
            pltpu.VMEM((rows_per_w, 64), jnp.float32),
            pltpu.SemaphoreType.DMA,
            pltpu.SemaphoreType.DMA,
            pltpu.SemaphoreType.DMA,
        ],
    )
    def gk(xf_hbm, cf_hbm, gidx_hbm, xo_hbm, co_hbm, tile_v, xr_v, cr_v,
           si, sx, sc):
        wid = lax.axis_index("s") * info.num_cores + lax.axis_index("c")
        icopies = []
        for h in range(16):
            icopies.append(pltpu.async_copy(
                gidx_hbm.at[pl.ds(h * 512 + wid * cols_per_w, cols_per_w)],
                tile_v.at[pl.ds(h * cols_per_w, cols_per_w)], si))
        for cp in icopies:
            cp.wait()
        copies = []
        for h in range(16):
            seg = tile_v.at[pl.ds(h * cols_per_w, cols_per_w)]
            copies.append(pltpu.async_copy(
                xf_hbm.at[seg], xr_v.at[pl.ds(h * cols_per_w, cols_per_w)],
                sx))
            copies.append(pltpu.async_copy(
                cf_hbm.at[seg], cr_v.at[pl.ds(h * cols_per_w, cols_per_w)],
                sc))
        for cp in copies:
            cp.wait()
        oiota = lax.iota(jnp.int32, 16) * 16 + wid * (cols_per_w * 16)
        ocopies = []
        for h in range(16):
            oidx = oiota + h
            ocopies.append(pltpu.async_copy(
                xr_v.at[pl.ds(h * cols_per_w, cols_per_w)], xo_hbm.at[oidx],
                sx))
            ocopies.append(pltpu.async_copy(
                cr_v.at[pl.ds(h * cols_per_w, cols_per_w)], co_hbm.at[oidx],
                sc))
        for cp in ocopies:
            cp.wait()

    return gk(xf, cf, gidx)


def kernel(x, coord, attn):
    b, l, c = x.shape
    nh, s = attn.shape[1], attn.shape[3]
    num_output = l // RATIO
    ch = c // nh

    gidx = _entropy_topk(attn.reshape(nh, l, s), num_output)
    gidx = gidx.reshape(nh * num_output)

    xf = x.reshape(l * nh, ch)
    cf = coord.reshape(l * nh, ch)
    xo, co = _sc_gather(xf, cf, gidx, num_output * nh)
    return (xo.reshape(b, num_output, c), co.reshape(b, num_output, c))


# speedup vs baseline: 1.2473x; 1.2473x over previous
import functools

import jax
import jax.numpy as jnp
from jax import lax
from jax.experimental import pallas as pl
from jax.experimental.pallas import tpu as pltpu
from jax.experimental.pallas import tpu_sc as plsc

RATIO = 4


def _entropy_topk_body(attn_ref, gidx_ref, ent_s, idx_s):
    h = pl.program_id(0)
    k = pl.program_id(1)
    nh = pl.num_programs(0) - 1
    nk = pl.num_programs(1)

    @pl.when(h < nh)
    def _entropy():
        a = attn_ref[...]
        e = (jnp.exp(a) * a)[0]
        lb, s = e.shape
        acc = e[:, 0:128]
        for t in range(1, s // 128):
            acc = acc + e[:, t * 128:(t + 1) * 128]
        tp = acc.T.reshape(16, 8, lb)
        p = tp[0]
        for t in range(1, 16):
            p = p + tp[t]
        t1 = p[0:4] + p[4:8]
        t2 = t1[0:2] + t1[2:4]
        v = t2[0:1] + t2[1:2]
        ent_s[pl.ds(h % 2, 1), pl.ds(k, 1), :] = v.reshape(1, 1, v.shape[-1])

    @pl.when(h >= 1)
    def _topk_chunk():
        hh = h - 1
        rm = (h + 1) % 2
        nck, lb = ent_s.shape[1], ent_s.shape[2]
        kk = gidx_ref.shape[-1]
        ci = k
        v8 = ent_s[pl.ds(rm, 1)][0]
        vi_row = ent_s[pl.ds(rm, 1), pl.ds(ci, 1), :].reshape(1, lb)
        vi_col = vi_row.reshape(lb, 1)
        cntmat = None
        jj = lax.broadcasted_iota(jnp.int32, (lb, lb), 1)
        ii = lax.broadcasted_iota(jnp.int32, (lb, lb), 0)
        intra = jj < ii
        for cj in range(nck):
            vj = v8[cj:cj + 1, :]
            gt = vj > vi_col
            eq = vj == vi_col
            w = jnp.where(cj < ci, 1.0, 0.0)
            mf = jnp.where(gt, 1.0, jnp.where(eq, w, 0.0))
            cntmat = mf if cntmat is None else cntmat + mf
        cntmat = cntmat + jnp.where((vi_row == vi_col) & intra, 1.0, 0.0)
        cnt = jnp.sum(cntmat, axis=1, keepdims=True)
        rr = lax.broadcasted_iota(jnp.int32, (lb, kk), 1).astype(jnp.float32)
        onehot = cnt == rr
        gi = (lax.broadcasted_iota(jnp.int32, (lb, kk), 0).astype(jnp.float32)
              + ci.astype(jnp.float32) * lb)
        contrib = jnp.sum(jnp.where(onehot, gi, 0.0), axis=0, keepdims=True)

        @pl.when(k == 0)
        def _():
            idx_s[...] = contrib

        @pl.when(k > 0)
        def _():
            idx_s[...] = idx_s[...] + contrib

        @pl.when(k == nk - 1)
        def _():
            gidx = idx_s[...].astype(jnp.int32) * 16 + hh
            gidx_ref[...] = gidx.reshape(gidx_ref.shape)


def _entropy_topk(attn3, num_output):
    nh, l, s = attn3.shape
    lb = 256
    nk = l // lb

    def attn_map(h, k):
        return (jnp.minimum(h, nh - 1), jnp.where(h == nh, nk - 1, k), 0)

    def out_map(h, k):
        return (jnp.maximum(h - 1, 0), 0, 0)

    return pl.pallas_call(
        _entropy_topk_body,
        grid=(nh + 1, nk),
        in_specs=[pl.BlockSpec((1, lb, s), attn_map)],
        out_specs=pl.BlockSpec((1, 1, num_output), out_map),
        out_shape=jax.ShapeDtypeStruct((nh, 1, num_output), jnp.int32),
        scratch_shapes=[
            pltpu.VMEM((2, nk, lb), jnp.float32),
            pltpu.VMEM((1, num_output), jnp.float32),
        ],
    )(attn3)


def _sc_gather(xf, cf, gidx, num_rows):
    mesh = plsc.VectorSubcoreMesh(core_axis_name="c", subcore_axis_name="s")
    info = plsc.get_sparse_core_info()
    nw = info.num_cores * info.num_subcores
    rows_per_w = num_rows // nw
    cols_per_w = rows_per_w // 16

    @functools.partial(
        pl.kernel,
        out_type=(
            jax.ShapeDtypeStruct((num_rows, 64), jnp.float32),
            jax.ShapeDtypeStruct((num_rows, 64), jnp.float32),
        ),
        mesh=mesh,
        compiler_params=pltpu.CompilerParams(use_tc_tiling_on_sc=False),
        scratch_types=[
            pltpu.VMEM((rows_per_w,), jnp.int32),
            pltpu.VMEM((rows_per_w, 64), jnp.float32),
            pltpu.VMEM((rows_per_w, 64), jnp.float32),
            pltpu.SemaphoreType.DMA,
            pltpu.SemaphoreType.DMA,
            pltpu.SemaphoreType.DMA,
        ],
    )
    def gk(xf_hbm, cf_hbm, gidx_hbm, xo_hbm, co_hbm, tile_v, xr_v, cr_v,
           si, sx, sc):
        wid = lax.axis_index("s") * info.num_cores + lax.axis_index("c")
        icopies = []
        for h in range(16):
            icopies.append(pltpu.async_copy(
                gidx_hbm.at[pl.ds(h * 512 + wid * cols_per_w, cols_per_w)],
                tile_v.at[pl.ds(h * cols_per_w, cols_per_w)], si))
        for cp in icopies:
            cp.wait()
        copies = []
        for h in range(16):
            seg = tile_v.at[pl.ds(h * cols_per_w, cols_per_w)]
            copies.append(pltpu.async_copy(
                xf_hbm.at[seg], xr_v.at[pl.ds(h * cols_per_w, cols_per_w)],
                sx))
            copies.append(pltpu.async_copy(
                cf_hbm.at[seg], cr_v.at[pl.ds(h * cols_per_w, cols_per_w)],
                sc))
        for cp in copies:
            cp.wait()
        oiota = lax.iota(jnp.int32, 16) * 16 + wid * (cols_per_w * 16)
        ocopies = []
        for h in range(16):
            oidx = oiota + h
            ocopies.append(pltpu.async_copy(
                xr_v.at[pl.ds(h * cols_per_w, cols_per_w)], xo_hbm.at[oidx],
                sx))
            ocopies.append(pltpu.async_copy(
                cr_v.at[pl.ds(h * cols_per_w, cols_per_w)], co_hbm.at[oidx],
                sc))
        for cp in ocopies:
            cp.wait()

    return gk(xf, cf, gidx)


def kernel(x, coord, attn):
    b, l, c = x.shape
    nh, s = attn.shape[1], attn.shape[3]
    num_output = l // RATIO
    ch = c // nh

    gidx = _entropy_topk(attn.reshape(nh, l, s), num_output)
    gidx = gidx.reshape(nh * num_output)

    xf = x.reshape(l * nh, ch)
    cf = coord.reshape(l * nh, ch)
    xo, co = _sc_gather(xf, cf, gidx, num_output * nh)
    return (xo.reshape(b, num_output, c), co.reshape(b, num_output, c))


# speedup vs baseline: 1.5262x; 1.2236x over previous
import functools

import jax
import jax.numpy as jnp
from jax import lax
from jax.experimental import pallas as pl
from jax.experimental.pallas import tpu as pltpu
from jax.experimental.pallas import tpu_sc as plsc

RATIO = 4


def _entropy_topk_body(attn_ref, gidx_ref, ent_s, idx_s):
    h = pl.program_id(0)
    k = pl.program_id(1)
    nh = pl.num_programs(0) - 1
    nk = pl.num_programs(1)

    @pl.when(h < nh)
    def _entropy():
        a = attn_ref[...]
        e = (jnp.exp(a) * a)[0]
        lb, s = e.shape
        acc = e[:, 0:128]
        for t in range(1, s // 128):
            acc = acc + e[:, t * 128:(t + 1) * 128]
        tp = acc.T.reshape(16, 8, lb)
        p = tp[0]
        for t in range(1, 16):
            p = p + tp[t]
        t1 = p[0:4] + p[4:8]
        t2 = t1[0:2] + t1[2:4]
        v = t2[0:1] + t2[1:2]
        ent_s[pl.ds(h % 2, 1), pl.ds(k, 1), :] = v.reshape(1, 1, v.shape[-1])

    @pl.when(h >= 1)
    def _topk_chunk():
        hh = h - 1
        rm = (h + 1) % 2
        nck, lb = ent_s.shape[1], ent_s.shape[2]
        kk = gidx_ref.shape[-1]
        ci = k
        v8 = ent_s[pl.ds(rm, 1)][0]
        vi_row = ent_s[pl.ds(rm, 1), pl.ds(ci, 1), :].reshape(1, lb)
        vi_col = vi_row.reshape(lb, 1)
        cntmat = None
        jj = lax.broadcasted_iota(jnp.int32, (lb, lb), 1)
        ii = lax.broadcasted_iota(jnp.int32, (lb, lb), 0)
        intra = jj < ii
        for cj in range(nck):
            vj = v8[cj:cj + 1, :]
            gt = vj > vi_col
            eq = vj == vi_col
            w = jnp.where(cj < ci, 1.0, 0.0)
            mf = jnp.where(gt, 1.0, jnp.where(eq, w, 0.0))
            cntmat = mf if cntmat is None else cntmat + mf
        cntmat = cntmat + jnp.where((vi_row == vi_col) & intra, 1.0, 0.0)
        cnt = jnp.sum(cntmat, axis=1, keepdims=True)
        rr = lax.broadcasted_iota(jnp.int32, (lb, kk), 1).astype(jnp.float32)
        onehot = cnt == rr
        gi = (lax.broadcasted_iota(jnp.int32, (lb, kk), 0).astype(jnp.float32)
              + ci.astype(jnp.float32) * lb)
        contrib = jnp.sum(jnp.where(onehot, gi, 0.0), axis=0, keepdims=True)

        @pl.when(k == 0)
        def _():
            idx_s[...] = contrib

        @pl.when(k > 0)
        def _():
            idx_s[...] = idx_s[...] + contrib

        @pl.when(k == nk - 1)
        def _():
            gidx = idx_s[...].astype(jnp.int32) * 16 + hh
            gidx_ref[...] = gidx.reshape(gidx_ref.shape)


def _entropy_topk(attn3, num_output):
    nh, l, s = attn3.shape
    lb = 512
    nk = l // lb

    def attn_map(h, k):
        return (jnp.minimum(h, nh - 1), jnp.where(h == nh, nk - 1, k), 0)

    def out_map(h, k):
        return (jnp.maximum(h - 1, 0), 0, 0)

    return pl.pallas_call(
        _entropy_topk_body,
        grid=(nh + 1, nk),
        in_specs=[pl.BlockSpec((1, lb, s), attn_map)],
        out_specs=pl.BlockSpec((1, 1, num_output), out_map),
        out_shape=jax.ShapeDtypeStruct((nh, 1, num_output), jnp.int32),
        scratch_shapes=[
            pltpu.VMEM((2, nk, lb), jnp.float32),
            pltpu.VMEM((1, num_output), jnp.float32),
        ],
    )(attn3)


def _sc_gather(xf, cf, gidx, num_rows):
    mesh = plsc.VectorSubcoreMesh(core_axis_name="c", subcore_axis_name="s")
    info = plsc.get_sparse_core_info()
    nw = info.num_cores * info.num_subcores
    rows_per_w = num_rows // nw
    cols_per_w = rows_per_w // 16

    @functools.partial(
        pl.kernel,
        out_type=(
            jax.ShapeDtypeStruct((num_rows, 64), jnp.float32),
            jax.ShapeDtypeStruct((num_rows, 64), jnp.float32),
        ),
        mesh=mesh,
        compiler_params=pltpu.CompilerParams(use_tc_tiling_on_sc=False),
        scratch_types=[
            pltpu.VMEM((rows_per_w,), jnp.int32),
            pltpu.VMEM((rows_per_w, 64), jnp.float32),
            pltpu.VMEM((rows_per_w, 64), jnp.float32),
            pltpu.SemaphoreType.DMA,
            pltpu.SemaphoreType.DMA,
            pltpu.SemaphoreType.DMA,
        ],
    )
    def gk(xf_hbm, cf_hbm, gidx_hbm, xo_hbm, co_hbm, tile_v, xr_v, cr_v,
           si, sx, sc):
        wid = lax.axis_index("s") * info.num_cores + lax.axis_index("c")
        icopies = []
        for h in range(16):
            icopies.append(pltpu.async_copy(
                gidx_hbm.at[pl.ds(h * 512 + wid * cols_per_w, cols_per_w)],
                tile_v.at[pl.ds(h * cols_per_w, cols_per_w)], si))
        for cp in icopies:
            cp.wait()
        copies = []
        for h in range(16):
            seg = tile_v.at[pl.ds(h * cols_per_w, cols_per_w)]
            copies.append(pltpu.async_copy(
                xf_hbm.at[seg], xr_v.at[pl.ds(h * cols_per_w, cols_per_w)],
                sx))
            copies.append(pltpu.async_copy(
                cf_hbm.at[seg], cr_v.at[pl.ds(h * cols_per_w, cols_per_w)],
                sc))
        for cp in copies:
            cp.wait()
        oiota = lax.iota(jnp.int32, 16) * 16 + wid * (cols_per_w * 16)
        ocopies = []
        for h in range(16):
            oidx = oiota + h
            ocopies.append(pltpu.async_copy(
                xr_v.at[pl.ds(h * cols_per_w, cols_per_w)], xo_hbm.at[oidx],
                sx))
            ocopies.append(pltpu.async_copy(
                cr_v.at[pl.ds(h * cols_per_w, cols_per_w)], co_hbm.at[oidx],
                sc))
        for cp in ocopies:
            cp.wait()

    return gk(xf, cf, gidx)


def kernel(x, coord, attn):
    b, l, c = x.shape
    nh, s = attn.shape[1], attn.shape[3]
    num_output = l // RATIO
    ch = c // nh

    gidx = _entropy_topk(attn.reshape(nh, l, s), num_output)
    gidx = gidx.reshape(nh * num_output)

    xf = x.reshape(l * nh, ch)
    cf = coord.reshape(l * nh, ch)
    xo, co = _sc_gather(xf, cf, gidx, num_output * nh)
    return (xo.reshape(b, num_output, c), co.reshape(b, num_output, c))


# speedup vs baseline: 1.6681x; 1.0930x over previous
import functools

import jax
import jax.numpy as jnp
from jax import lax
from jax.experimental import pallas as pl
from jax.experimental.pallas import tpu as pltpu
from jax.experimental.pallas import tpu_sc as plsc

RATIO = 4


def _entropy_topk_body(attn_ref, gidx_ref, ent_s, idx_s):
    h = pl.program_id(0)
    k = pl.program_id(1)
    nh = pl.num_programs(0) - 1
    nk = pl.num_programs(1)

    @pl.when(h < nh)
    def _entropy():
        a = attn_ref[...]
        e = (jnp.exp(a) * a)[0]
        lb, s = e.shape
        acc = e[:, 0:128]
        for t in range(1, s // 128):
            acc = acc + e[:, t * 128:(t + 1) * 128]
        tp = acc.T.reshape(16, 8, lb)
        p = tp[0]
        for t in range(1, 16):
            p = p + tp[t]
        t1 = p[0:4] + p[4:8]
        t2 = t1[0:2] + t1[2:4]
        v = t2[0:1] + t2[1:2]
        ent_s[pl.ds(h % 2, 1), pl.ds(k, 1), :] = v.reshape(1, 1, v.shape[-1])

    @pl.when(h >= 1)
    def _topk_chunk():
        hh = h - 1
        rm = (h + 1) % 2
        nck, lb = ent_s.shape[1], ent_s.shape[2]
        kk = gidx_ref.shape[-1]
        ci = k
        v8 = ent_s[pl.ds(rm, 1)][0]
        vi_row = ent_s[pl.ds(rm, 1), pl.ds(ci, 1), :].reshape(1, lb)
        vi_col = vi_row.reshape(lb, 1)
        cntmat = None
        jj = lax.broadcasted_iota(jnp.int32, (lb, lb), 1)
        ii = lax.broadcasted_iota(jnp.int32, (lb, lb), 0)
        intra = jj < ii
        for cj in range(nck):
            vj = v8[cj:cj + 1, :]
            gt = vj > vi_col
            eq = vj == vi_col
            w = jnp.where(cj < ci, 1.0, 0.0)
            mf = jnp.where(gt, 1.0, jnp.where(eq, w, 0.0))
            cntmat = mf if cntmat is None else cntmat + mf
        cntmat = cntmat + jnp.where((vi_row == vi_col) & intra, 1.0, 0.0)
        cnt = jnp.sum(cntmat, axis=1, keepdims=True)
        rr = lax.broadcasted_iota(jnp.int32, (lb, kk), 1).astype(jnp.float32)
        onehot = cnt == rr
        gi = (lax.broadcasted_iota(jnp.int32, (lb, kk), 0).astype(jnp.float32)
              + ci.astype(jnp.float32) * lb)
        contrib = jnp.sum(jnp.where(onehot, gi, 0.0), axis=0, keepdims=True)

        @pl.when(k == 0)
        def _():
            idx_s[...] = contrib

        @pl.when(k > 0)
        def _():
            idx_s[...] = idx_s[...] + contrib

        @pl.when(k == nk - 1)
        def _():
            gidx = idx_s[...].astype(jnp.int32) * 16 + hh
            gidx_ref[...] = gidx.reshape(gidx_ref.shape)


def _entropy_topk(attn3, num_output):
    nh, l, s = attn3.shape
    lb = 1024
    nk = l // lb

    def attn_map(h, k):
        return (jnp.minimum(h, nh - 1), jnp.where(h == nh, nk - 1, k), 0)

    def out_map(h, k):
        return (jnp.maximum(h - 1, 0), 0, 0)

    return pl.pallas_call(
        _entropy_topk_body,
        grid=(nh + 1, nk),
        in_specs=[pl.BlockSpec((1, lb, s), attn_map)],
        out_specs=pl.BlockSpec((1, 1, num_output), out_map),
        out_shape=jax.ShapeDtypeStruct((nh, 1, num_output), jnp.int32),
        scratch_shapes=[
            pltpu.VMEM((2, nk, lb), jnp.float32),
            pltpu.VMEM((1, num_output), jnp.float32),
        ],
    )(attn3)


def _sc_gather(xf, cf, gidx, num_rows):
    mesh = plsc.VectorSubcoreMesh(core_axis_name="c", subcore_axis_name="s")
    info = plsc.get_sparse_core_info()
    nw = info.num_cores * info.num_subcores
    rows_per_w = num_rows // nw
    cols_per_w = rows_per_w // 16

    @functools.partial(
        pl.kernel,
        out_type=(
            jax.ShapeDtypeStruct((num_rows, 64), jnp.float32),
            jax.ShapeDtypeStruct((num_rows, 64), jnp.float32),
        ),
        mesh=mesh,
        compiler_params=pltpu.CompilerParams(use_tc_tiling_on_sc=False),
        scratch_types=[
            pltpu.VMEM((rows_per_w,), jnp.int32),
            pltpu.VMEM((rows_per_w, 64), jnp.float32),
            pltpu.VMEM((rows_per_w, 64), jnp.float32),
            pltpu.SemaphoreType.DMA,
            pltpu.SemaphoreType.DMA,
            pltpu.SemaphoreType.DMA,
        ],
    )
    def gk(xf_hbm, cf_hbm, gidx_hbm, xo_hbm, co_hbm, tile_v, xr_v, cr_v,
           si, sx, sc):
        wid = lax.axis_index("s") * info.num_cores + lax.axis_index("c")
        icopies = []
        for h in range(16):
            icopies.append(pltpu.async_copy(
                gidx_hbm.at[pl.ds(h * 512 + wid * cols_per_w, cols_per_w)],
                tile_v.at[pl.ds(h * cols_per_w, cols_per_w)], si))
        for cp in icopies:
            cp.wait()
        copies = []
        for h in range(16):
            seg = tile_v.at[pl.ds(h * cols_per_w, cols_per_w)]
            copies.append(pltpu.async_copy(
                xf_hbm.at[seg], xr_v.at[pl.ds(h * cols_per_w, cols_per_w)],
                sx))
            copies.append(pltpu.async_copy(
                cf_hbm.at[seg], cr_v.at[pl.ds(h * cols_per_w, cols_per_w)],
                sc))
        for cp in copies:
            cp.wait()
        oiota = lax.iota(jnp.int32, 16) * 16 + wid * (cols_per_w * 16)
        ocopies = []
        for h in range(16):
            oidx = oiota + h
            ocopies.append(pltpu.async_copy(
                xr_v.at[pl.ds(h * cols_per_w, cols_per_w)], xo_hbm.at[oidx],
                sx))
            ocopies.append(pltpu.async_copy(
                cr_v.at[pl.ds(h * cols_per_w, cols_per_w)], co_hbm.at[oidx],
                sc))
        for cp in ocopies:
            cp.wait()

    return gk(xf, cf, gidx)


def kernel(x, coord, attn):
    b, l, c = x.shape
    nh, s = attn.shape[1], attn.shape[3]
    num_output = l // RATIO
    ch = c // nh

    gidx = _entropy_topk(attn.reshape(nh, l, s), num_output)
    gidx = gidx.reshape(nh * num_output)

    xf = x.reshape(l * nh, ch)
    cf = coord.reshape(l * nh, ch)
    xo, co = _sc_gather(xf, cf, gidx, num_output * nh)
    return (xo.reshape(b, num_output, c), co.reshape(b, num_output, c))
